# baseline (speedup 1.0000x reference)
import math

import jax
import jax.numpy as jnp
from jax import lax
from jax.experimental import pallas as pl
from jax.experimental.pallas import tpu as pltpu
from jax.experimental.pallas import tpu_sc as plsc

EMB = 64
SCALE = math.sqrt(EMB)
SEQ = 200
NBUF = 2
SPLITS = ((0, 128), (128, 72))
IDXPAD = 16


def kernel(tokens, table):
    nbatch, seq = tokens.shape
    assert seq == SEQ
    B = nbatch * seq
    vocab = table.shape[0]
    info = plsc.get_sparse_core_info()
    n_workers = info.num_cores * info.num_subcores
    rows_per_w = nbatch // n_workers
    toks_per_w = rows_per_w * seq
    mesh = plsc.VectorSubcoreMesh(core_axis_name="c", subcore_axis_name="s")

    def body(tokens_hbm, table_hbm, out_hbm, idx_all, idx2, rows, rows_o, sg, ss):
        wid = lax.axis_index("s") * info.num_cores + lax.axis_index("c")
        wrow0 = wid * rows_per_w

        pltpu.sync_copy(
            tokens_hbm.at[pl.ds(wid * toks_per_w, toks_per_w)],
            idx_all.at[pl.ds(0, toks_per_w)],
        )

        def fire(ci, b):
            base = ci * SEQ
            for j in range((SEQ + 15) // 16):
                sl = pl.ds(j * 16, 16)
                idx2[b][sl] = lax.shift_right_logical(idx_all[pl.ds(base + j * 16, 16)], 1)
            for (off, n) in SPLITS:
                pltpu.async_copy(
                    table_hbm.at[idx2[b].at[pl.ds(off, n)]],
                    rows[b].at[pl.ds(off, n)],
                    sg[b],
                )

        def wait_gather(b):
            for (off, n) in SPLITS:
                pltpu.make_async_copy(
                    table_hbm.at[idx2[b].at[pl.ds(off, n)]],
                    rows[b].at[pl.ds(off, n)],
                    sg[b],
                ).wait()

        def store(ci, b):
            pltpu.async_copy(rows_o[b], out_hbm.at[wrow0 + ci], ss[b])

        def wait_store(b):
            pltpu.make_async_copy(rows_o[b], out_hbm.at[wrow0], ss[b]).wait()

        def scale(ci, b):
            base = ci * SEQ

            def scale_block(k, c):
                r0 = k * 8
                par_vec = (idx_all[pl.ds(base + r0, 16)] & 1) * EMB
                for j in range(8):
                    par = par_vec[j]
                    for q in range(EMB // 16):
                        v = rows[b][r0 + j, pl.ds(par + q * 16, 16)]
                        rows_o[b][r0 + j, pl.ds(q * 16, 16)] = v * SCALE
                return c

            lax.fori_loop(0, SEQ // 8, scale_block, 0)

        fire(0, 0)

        def step(ci, carry):
            for b in range(NBUF):
                i = ci * NBUF + b
                nb = (b + 1) % NBUF
                @pl.when(i >= 1)
                def _():
                    wait_store(nb)
                @pl.when(i + 1 < rows_per_w)
                def _():
                    fire(i + 1, nb)
                wait_gather(b)
                scale(i, b)
                store(i, b)
            return carry

        lax.fori_loop(0, rows_per_w // NBUF, step, 0)
        wait_store((rows_per_w - 1) % NBUF)

    return pl.kernel(
        body,
        out_type=jax.ShapeDtypeStruct((nbatch, seq, EMB), jnp.float32),
        mesh=mesh,
        scratch_types=[
            pltpu.VMEM((toks_per_w + IDXPAD,), jnp.int32),
            [pltpu.VMEM((SEQ + IDXPAD,), jnp.int32) for _ in range(NBUF)],
            [pltpu.VMEM((SEQ, 2 * EMB), jnp.float32) for _ in range(NBUF)],
            [pltpu.VMEM((SEQ, EMB), jnp.float32) for _ in range(NBUF)],
            [pltpu.SemaphoreType.DMA for _ in range(NBUF)],
            [pltpu.SemaphoreType.DMA for _ in range(NBUF)],
        ],
    )(tokens.reshape(B), table.reshape(vocab // 2, 2 * EMB))
